# baseline (speedup 1.0000x reference)
import functools

import jax
import jax.numpy as jnp
from jax import lax
from jax.experimental import pallas as pl
from jax.experimental.pallas import tpu as pltpu
from jax.experimental.pallas import tpu_sc as plsc

_NC = 2
_NS = 16
_NW = _NC * _NS
_L = 200
_LP = 256
_RB = 32


def _dg(vec, idx):
    return vec.at[idx].get(mode="promise_in_bounds")


def _body(idx_hbm, par_hbm, out_hbm, par_v,
          idx_v0, idx_v1, out_v0, out_v1, s_i0, s_i1, s_o0, s_o1):
    wid = lax.axis_index("s") * _NC + lax.axis_index("c")
    rows_per_w = idx_hbm.shape[0] // (_L * _NW)
    nblk = rows_per_w // _RB

    pltpu.sync_copy(par_hbm, par_v)
    lane = lax.iota(jnp.int32, 16)
    emb_v = par_v[pl.ds(0, 16)]
    w_v = par_v[pl.ds(16, 16)]
    b_v = par_v[pl.ds(32, 16)]
    k2 = jnp.minimum(lane, 7) * 2
    luts = []
    for c in range(4):
        cc = jnp.full((16,), c, jnp.int32)
        luts.append(_dg(emb_v, k2) * _dg(w_v, cc)
                    + _dg(emb_v, k2 + 1) * _dg(w_v, cc + 4)
                    + _dg(b_v, cc))

    idx_bufs = [idx_v0, idx_v1]
    out_bufs = [out_v0, out_v1]
    si = [s_i0, s_i1]
    so = [s_o0, s_o1]
    ibase = wid * rows_per_w * _L
    obase = wid * rows_per_w * _LP * 4

    icopy = [None, None]
    ocopy = [None, None]
    icopy[0] = pltpu.async_copy(idx_hbm.at[pl.ds(ibase, _RB * _L)],
                                idx_bufs[0].at[pl.ds(0, _RB * _L)], si[0])
    for t in range(nblk):
        cur = t & 1
        icopy[cur].wait()
        if t + 1 < nblk:
            icopy[1 - cur] = pltpu.async_copy(
                idx_hbm.at[pl.ds(ibase + (t + 1) * _RB * _L, _RB * _L)],
                idx_bufs[1 - cur].at[pl.ds(0, _RB * _L)], si[1 - cur])
        if t >= 2:
            ocopy[cur].wait()
        iv = idx_bufs[cur]
        ov = out_bufs[cur]

        def row_body(r, carry, iv=iv, ov=ov):
            for c in range(4):
                for j in range(8):
                    v = iv[pl.ds(r * _L + j * 16, 16)]
                    ov[pl.ds(r * _LP * 4 + c * 128 + j * 16, 16)] = \
                        _dg(luts[c], v)
                for m in range(5):
                    v = iv[pl.ds(r * _L + 128 + m * 16, 16)]
                    ov[pl.ds(r * _LP * 4 + 512 + c * 128 + m * 16, 16)] = \
                        _dg(luts[c], v)
            return carry

        lax.fori_loop(0, _RB, row_body, 0)
        ocopy[cur] = pltpu.async_copy(
            ov.at[pl.ds(0, _RB * _LP * 4)],
            out_hbm.at[pl.ds(obase + t * _RB * _LP * 4, _RB * _LP * 4)],
            so[cur])
    ocopy[0].wait()
    ocopy[1].wait()


def kernel(inputs, emb_table, W, b):
    B, L = inputs.shape
    N = B * L
    fone = W[0, 0] * 0.0 + 1.0
    izero = fone.astype(jnp.int32) - 1
    idx_flat = inputs.reshape(N).astype(jnp.int32) + izero
    par = jnp.zeros((48,), jnp.float32)
    par = par.at[0:6].set(emb_table.reshape(-1))
    par = par.at[16:24].set(W.reshape(-1))
    par = par.at[32:36].set(b)

    mesh = plsc.VectorSubcoreMesh(core_axis_name="c", subcore_axis_name="s")
    run = functools.partial(
        pl.kernel,
        mesh=mesh,
        out_type=jax.ShapeDtypeStruct((B * _LP * 4,), jnp.float32),
        scratch_types=[
            pltpu.VMEM((48,), jnp.float32),
            pltpu.VMEM((_RB * _L + 8,), jnp.int32),
            pltpu.VMEM((_RB * _L + 8,), jnp.int32),
            pltpu.VMEM((_RB * _LP * 4,), jnp.float32),
            pltpu.VMEM((_RB * _LP * 4,), jnp.float32),
            pltpu.SemaphoreType.DMA,
            pltpu.SemaphoreType.DMA,
            pltpu.SemaphoreType.DMA,
            pltpu.SemaphoreType.DMA,
        ],
    )(_body)
    out = run(idx_flat, par)
    y = out.reshape(B, 2, 4, 128).transpose(0, 1, 3, 2).reshape(B, _LP, 4)
    return y[:, :L, :] * fone
